# baseline (speedup 1.0000x reference)
import jax
import jax.numpy as jnp
from jax import lax
from jax.experimental import pallas as pl
from jax.experimental.pallas import tpu as pltpu
from jax.experimental.pallas import tpu_sc as plsc

NCLS = 100
FDIM = 512
BATCH_ = 4096
LAMDA_ = 0.5
EPS = 1e-8

RT = 3584
TBLK = 512

NC = 2
NS = 16
NW = NC * NS
LANES = 16
NCG = 4
CGW = FDIM // NCG
NRG = NW // NCG
SCROWS = BATCH_ - RT
RGH = SCROWS // NRG
KCH = CGW // LANES
RPB = 2
SPAD = 104


def _sc_body(label_ref, feat_ref, s_out, a_out,
             lblbuf, fbuf, abuf, s_local, sem0, sem1):
    c = lax.axis_index("c")
    s = lax.axis_index("s")
    wid = s * NC + c
    cg = lax.rem(wid, NCG)
    rg = lax.div(wid, NCG)
    row0 = RT + rg * RGH
    col0 = cg * CGW

    half = RGH // 2
    d0 = pltpu.async_copy(
        feat_ref.at[pl.ds(row0, half), pl.ds(col0, CGW)],
        fbuf.at[pl.ds(0, half)], sem0)
    d1 = pltpu.async_copy(
        feat_ref.at[pl.ds(row0 + half, half), pl.ds(col0, CGW)],
        fbuf.at[pl.ds(half, half)], sem1)
    pltpu.sync_copy(label_ref.at[pl.ds(row0, RGH)], lblbuf.at[pl.ds(0, RGH)])

    zero = jnp.zeros((LANES,), jnp.float32)

    def zero_body(i, carry):
        for r in range(4):
            for k in range(KCH):
                s_local[i * 4 + r, pl.ds(k * LANES, LANES)] = zero
        return carry

    lax.fori_loop(0, SPAD // 4, zero_body, 0)

    nacc = 8
    accs0 = tuple(jnp.zeros((LANES,), jnp.float32) for _ in range(nacc))

    def make_loop(lo, hi, accs_in):
        @plsc.parallel_loop(lo, hi, step=RPB, carry=accs_in)
        def loop(i, accs):
            lblv = lblbuf[pl.ds(i, LANES)]
            vs = [[fbuf[i + r, pl.ds(k * LANES, LANES)] for k in range(KCH)]
                  for r in range(RPB)]
            accs = list(accs)
            for r in range(RPB):
                l = lblv[r]
                for k in range(KCH):
                    plsc.addupdate(s_local.at[l, pl.ds(k * LANES, LANES)],
                                   vs[r][k])
                    j = (r * KCH + k) % nacc
                    accs[j] = accs[j] + vs[r][k] * vs[r][k]
            return tuple(accs)
        return loop

    d0.wait()
    accs = make_loop(0, half, accs0)
    d1.wait()
    accs = make_loop(half, RGH, accs)

    tot = accs[0]
    for t in accs[1:]:
        tot = tot + t
    abuf[0, pl.ds(0, LANES)] = tot
    pltpu.sync_copy(abuf, a_out.at[wid])
    pltpu.sync_copy(s_local, s_out.at[cg * NRG + rg])


def _sc_stage(label, feat):
    mesh = plsc.VectorSubcoreMesh(core_axis_name="c", subcore_axis_name="s")
    return pl.kernel(
        _sc_body,
        out_type=(
            jax.ShapeDtypeStruct((NW, SPAD, CGW), jnp.float32),
            jax.ShapeDtypeStruct((NW, 1, LANES), jnp.float32),
        ),
        mesh=mesh,
        scratch_types=[
            pltpu.VMEM((RGH + LANES,), jnp.int32),
            pltpu.VMEM((RGH, CGW), jnp.float32),
            pltpu.VMEM((1, LANES), jnp.float32),
            pltpu.VMEM((SPAD, CGW), jnp.float32),
            pltpu.SemaphoreType.DMA,
            pltpu.SemaphoreType.DMA,
        ],
    )(label, feat)


def _tc_main_body(label_ref, feat_ref, centers_ref, out_ref,
                  s_acc, cnt_acc, a_acc):
    i = pl.program_id(0)
    nsteps = pl.num_programs(0)

    @pl.when(i == 0)
    def _init():
        s_acc[...] = jnp.zeros_like(s_acc)
        cnt_acc[...] = jnp.zeros_like(cnt_acc)
        a_acc[...] = jnp.zeros_like(a_acc)

    feat = feat_ref[...]
    lbl = label_ref[0, 0, :]
    onehot = (lbl[:, None] == lax.broadcasted_iota(jnp.int32, (1, NCLS), 1)
              ).astype(jnp.float32)
    s_acc[...] += lax.dot_general(onehot, feat, (((0,), (0,)), ((), ())),
                                  preferred_element_type=jnp.float32)
    cnt_acc[...] += jnp.sum(onehot, axis=0, keepdims=True)
    a_acc[...] += jnp.sum(feat * feat, axis=0, keepdims=True)

    @pl.when(i == nsteps - 1)
    def _fini():
        centers = centers_ref[...]
        a = jnp.sum(a_acc[...])
        b = jnp.sum(s_acc[...] * centers)
        n2 = jnp.sum(centers * centers, axis=1)
        csum = jnp.sum(cnt_acc[0, :] * n2)
        inv = 1.0 / jnp.maximum(jnp.sqrt(n2), EPS)
        cn = centers * inv[:, None]
        s_vec = jnp.sum(cn, axis=0)
        island = (jnp.sum(s_vec * s_vec) - jnp.sum(cn * cn)
                  + float(NCLS * NCLS - NCLS))
        part = (a - 2.0 * b + csum
                + 2.0 * BATCH_ * LAMDA_ * island)
        out_ref[...] = jnp.reshape(part, (1, 1))


def _tc_main(label3, feat, centers):
    return pl.pallas_call(
        _tc_main_body,
        grid=(RT // TBLK,),
        in_specs=[
            pl.BlockSpec((1, 1, TBLK), lambda i: (i, 0, 0)),
            pl.BlockSpec((TBLK, FDIM), lambda i: (i, 0)),
            pl.BlockSpec((NCLS, FDIM), lambda i: (0, 0)),
        ],
        out_specs=pl.BlockSpec((1, 1), lambda i: (0, 0)),
        out_shape=jax.ShapeDtypeStruct((1, 1), jnp.float32),
        scratch_shapes=[
            pltpu.VMEM((NCLS, FDIM), jnp.float32),
            pltpu.VMEM((1, NCLS), jnp.float32),
            pltpu.VMEM((1, FDIM), jnp.float32),
        ],
    )(label3, feat, centers)


def _tc_epi_body(s_ref, a_ref, label_ref, centers_ref, part_ref, out_ref):
    centers = centers_ref[...]
    a = jnp.sum(a_ref[...])
    b = jnp.float32(0.0)
    for cg in range(NCG):
        scg = s_ref[cg * NRG + 0, :NCLS, :]
        for rg in range(1, NRG):
            scg = scg + s_ref[cg * NRG + rg, :NCLS, :]
        b += jnp.sum(scg * centers[:, cg * CGW:(cg + 1) * CGW])
    n2 = jnp.sum(centers * centers, axis=1)
    cnt = jnp.zeros((1, NCLS), jnp.float32)
    iota = lax.broadcasted_iota(jnp.int32, (1, NCLS), 1)
    for r in range(RT // TBLK, BATCH_ // TBLK):
        lbl = label_ref[r, 0, :]
        cnt += jnp.sum((lbl[:, None] == iota).astype(jnp.float32),
                       axis=0, keepdims=True)
    csum = jnp.sum(cnt[0, :] * n2)
    total = part_ref[0, 0] + a - 2.0 * b + csum
    out_ref[...] = jnp.reshape(total / 2.0 / BATCH_, (1, 1))


def _tc_epilogue(s_cols, a_part, label3, centers, tc_part):
    return pl.pallas_call(
        _tc_epi_body,
        grid=(1,),
        in_specs=[
            pl.BlockSpec((NW, SPAD, CGW), lambda i: (0, 0, 0)),
            pl.BlockSpec((NW, 1, LANES), lambda i: (0, 0, 0)),
            pl.BlockSpec((BATCH_ // TBLK, 1, TBLK), lambda i: (0, 0, 0)),
            pl.BlockSpec((NCLS, FDIM), lambda i: (0, 0)),
            pl.BlockSpec((1, 1), lambda i: (0, 0)),
        ],
        out_specs=pl.BlockSpec((1, 1), lambda i: (0, 0)),
        out_shape=jax.ShapeDtypeStruct((1, 1), jnp.float32),
    )(s_cols, a_part, label3, centers, tc_part)


def kernel(label, feat, centers):
    label3 = label.reshape(BATCH_ // TBLK, 1, TBLK)
    tc_part = _tc_main(label3, feat, centers)
    s_cols, a_part = _sc_stage(label, feat)
    out = _tc_epilogue(s_cols, a_part, label3, centers, tc_part)
    return out.reshape(1)


# speedup vs baseline: 1.0203x; 1.0203x over previous
import jax
import jax.numpy as jnp
from jax import lax
from jax.experimental import pallas as pl
from jax.experimental.pallas import tpu as pltpu
from jax.experimental.pallas import tpu_sc as plsc

NCLS = 100
FDIM = 512
BATCH_ = 4096
LAMDA_ = 0.5
EPS = 1e-8

RT = 3072
TBLK = 512

NC = 2
NS = 16
NW = NC * NS
LANES = 16
NCG = 4
CGW = FDIM // NCG
NRG = NW // NCG
SCROWS = BATCH_ - RT
RGH = SCROWS // NRG
KCH = CGW // LANES
RPB = 2
SPAD = 104


def _sc_body(label_ref, feat_ref, s_out, a_out,
             lblbuf, fbuf, abuf, s_local, sem0, sem1):
    c = lax.axis_index("c")
    s = lax.axis_index("s")
    wid = s * NC + c
    cg = lax.rem(wid, NCG)
    rg = lax.div(wid, NCG)
    row0 = RT + rg * RGH
    col0 = cg * CGW

    half = RGH // 2
    d0 = pltpu.async_copy(
        feat_ref.at[pl.ds(row0, half), pl.ds(col0, CGW)],
        fbuf.at[pl.ds(0, half)], sem0)
    d1 = pltpu.async_copy(
        feat_ref.at[pl.ds(row0 + half, half), pl.ds(col0, CGW)],
        fbuf.at[pl.ds(half, half)], sem1)
    pltpu.sync_copy(label_ref.at[pl.ds(row0, RGH)], lblbuf.at[pl.ds(0, RGH)])

    zero = jnp.zeros((LANES,), jnp.float32)

    def zero_body(i, carry):
        for r in range(4):
            for k in range(KCH):
                s_local[i * 4 + r, pl.ds(k * LANES, LANES)] = zero
        return carry

    lax.fori_loop(0, SPAD // 4, zero_body, 0)

    nacc = 8
    accs0 = tuple(jnp.zeros((LANES,), jnp.float32) for _ in range(nacc))

    def make_loop(lo, hi, accs_in):
        @plsc.parallel_loop(lo, hi, step=RPB, carry=accs_in)
        def loop(i, accs):
            lblv = lblbuf[pl.ds(i, LANES)]
            vs = [[fbuf[i + r, pl.ds(k * LANES, LANES)] for k in range(KCH)]
                  for r in range(RPB)]
            accs = list(accs)
            for r in range(RPB):
                l = lblv[r]
                for k in range(KCH):
                    plsc.addupdate(s_local.at[l, pl.ds(k * LANES, LANES)],
                                   vs[r][k])
                    j = (r * KCH + k) % nacc
                    accs[j] = accs[j] + vs[r][k] * vs[r][k]
            return tuple(accs)
        return loop

    d0.wait()
    accs = make_loop(0, half, accs0)
    d1.wait()
    accs = make_loop(half, RGH, accs)

    tot = accs[0]
    for t in accs[1:]:
        tot = tot + t
    abuf[0, pl.ds(0, LANES)] = tot
    pltpu.sync_copy(abuf, a_out.at[wid])
    pltpu.sync_copy(s_local, s_out.at[cg * NRG + rg])


def _sc_stage(label, feat):
    mesh = plsc.VectorSubcoreMesh(core_axis_name="c", subcore_axis_name="s")
    return pl.kernel(
        _sc_body,
        out_type=(
            jax.ShapeDtypeStruct((NW, SPAD, CGW), jnp.float32),
            jax.ShapeDtypeStruct((NW, 1, LANES), jnp.float32),
        ),
        mesh=mesh,
        scratch_types=[
            pltpu.VMEM((RGH + LANES,), jnp.int32),
            pltpu.VMEM((RGH, CGW), jnp.float32),
            pltpu.VMEM((1, LANES), jnp.float32),
            pltpu.VMEM((SPAD, CGW), jnp.float32),
            pltpu.SemaphoreType.DMA,
            pltpu.SemaphoreType.DMA,
        ],
    )(label, feat)


def _tc_main_body(label_ref, feat_ref, centers_ref, out_ref,
                  s_acc, cnt_acc, a_acc):
    i = pl.program_id(0)
    nsteps = pl.num_programs(0)

    @pl.when(i == 0)
    def _init():
        s_acc[...] = jnp.zeros_like(s_acc)
        cnt_acc[...] = jnp.zeros_like(cnt_acc)
        a_acc[...] = jnp.zeros_like(a_acc)

    feat = feat_ref[...]
    lbl = label_ref[0, 0, :]
    onehot = (lbl[:, None] == lax.broadcasted_iota(jnp.int32, (1, NCLS), 1)
              ).astype(jnp.float32)
    s_acc[...] += lax.dot_general(onehot, feat, (((0,), (0,)), ((), ())),
                                  preferred_element_type=jnp.float32)
    cnt_acc[...] += jnp.sum(onehot, axis=0, keepdims=True)
    a_acc[...] += jnp.sum(feat * feat, axis=0, keepdims=True)

    @pl.when(i == nsteps - 1)
    def _fini():
        centers = centers_ref[...]
        a = jnp.sum(a_acc[...])
        b = jnp.sum(s_acc[...] * centers)
        n2 = jnp.sum(centers * centers, axis=1)
        csum = jnp.sum(cnt_acc[0, :] * n2)
        inv = 1.0 / jnp.maximum(jnp.sqrt(n2), EPS)
        cn = centers * inv[:, None]
        s_vec = jnp.sum(cn, axis=0)
        island = (jnp.sum(s_vec * s_vec) - jnp.sum(cn * cn)
                  + float(NCLS * NCLS - NCLS))
        part = (a - 2.0 * b + csum
                + 2.0 * BATCH_ * LAMDA_ * island)
        out_ref[...] = jnp.reshape(part, (1, 1))


def _tc_main(label3, feat, centers):
    return pl.pallas_call(
        _tc_main_body,
        grid=(RT // TBLK,),
        in_specs=[
            pl.BlockSpec((1, 1, TBLK), lambda i: (i, 0, 0)),
            pl.BlockSpec((TBLK, FDIM), lambda i: (i, 0)),
            pl.BlockSpec((NCLS, FDIM), lambda i: (0, 0)),
        ],
        out_specs=pl.BlockSpec((1, 1), lambda i: (0, 0)),
        out_shape=jax.ShapeDtypeStruct((1, 1), jnp.float32),
        scratch_shapes=[
            pltpu.VMEM((NCLS, FDIM), jnp.float32),
            pltpu.VMEM((1, NCLS), jnp.float32),
            pltpu.VMEM((1, FDIM), jnp.float32),
        ],
    )(label3, feat, centers)


def _tc_epi_body(s_ref, a_ref, label_ref, centers_ref, part_ref, out_ref):
    centers = centers_ref[...]
    a = jnp.sum(a_ref[...])
    b = jnp.float32(0.0)
    for cg in range(NCG):
        scg = s_ref[cg * NRG + 0, :NCLS, :]
        for rg in range(1, NRG):
            scg = scg + s_ref[cg * NRG + rg, :NCLS, :]
        b += jnp.sum(scg * centers[:, cg * CGW:(cg + 1) * CGW])
    n2 = jnp.sum(centers * centers, axis=1)
    cnt = jnp.zeros((1, NCLS), jnp.float32)
    iota = lax.broadcasted_iota(jnp.int32, (1, NCLS), 1)
    for r in range(RT // TBLK, BATCH_ // TBLK):
        lbl = label_ref[r, 0, :]
        cnt += jnp.sum((lbl[:, None] == iota).astype(jnp.float32),
                       axis=0, keepdims=True)
    csum = jnp.sum(cnt[0, :] * n2)
    total = part_ref[0, 0] + a - 2.0 * b + csum
    out_ref[...] = jnp.reshape(total / 2.0 / BATCH_, (1, 1))


def _tc_epilogue(s_cols, a_part, label3, centers, tc_part):
    return pl.pallas_call(
        _tc_epi_body,
        grid=(1,),
        in_specs=[
            pl.BlockSpec((NW, SPAD, CGW), lambda i: (0, 0, 0)),
            pl.BlockSpec((NW, 1, LANES), lambda i: (0, 0, 0)),
            pl.BlockSpec((BATCH_ // TBLK, 1, TBLK), lambda i: (0, 0, 0)),
            pl.BlockSpec((NCLS, FDIM), lambda i: (0, 0)),
            pl.BlockSpec((1, 1), lambda i: (0, 0)),
        ],
        out_specs=pl.BlockSpec((1, 1), lambda i: (0, 0)),
        out_shape=jax.ShapeDtypeStruct((1, 1), jnp.float32),
    )(s_cols, a_part, label3, centers, tc_part)


def kernel(label, feat, centers):
    label3 = label.reshape(BATCH_ // TBLK, 1, TBLK)
    tc_part = _tc_main(label3, feat, centers)
    s_cols, a_part = _sc_stage(label, feat)
    out = _tc_epilogue(s_cols, a_part, label3, centers, tc_part)
    return out.reshape(1)
